# baseline (speedup 1.0000x reference)
import functools

import jax
import jax.numpy as jnp
from jax import lax
from jax.experimental import pallas as pl
from jax.experimental.pallas import tpu as pltpu
from jax.experimental.pallas import tpu_sc as plsc

F32 = jnp.float32
_EPS = 1e-5

_BCAST_DNUMS = lax.GatherDimensionNumbers(
    offset_dims=(), collapsed_slice_dims=(0,), start_index_map=(0,))


def _bcast16(v, lane):
    idx = jnp.full((16, 1), lane, jnp.int32)
    return lax.gather(v, idx, _BCAST_DNUMS, (1,),
                      indices_are_sorted=True, unique_indices=False,
                      mode=lax.GatherScatterMode.PROMISE_IN_BOUNDS)


def _softplus(x):
    return jnp.maximum(x, 0.0) + jnp.log1p(jnp.exp(-jnp.abs(x)))


def _silu(x):
    return x * jax.nn.sigmoid(x)


def _stage_to_spmem(table_h, shared, buf, sid, chunks, rows):
    for c in range(chunks):
        base = sid * (chunks * rows) + c * rows
        pltpu.sync_copy(table_h.at[pl.ds(base, rows)], buf)
        pltpu.sync_copy(buf, shared.at[pl.ds(base, rows)])
    plsc.subcore_barrier()


def _sc1_call(table, idx_r, w_r, npad, rpw, steps):
    A = table.shape[1]
    NB = 2
    EW = 128
    mesh = plsc.VectorSubcoreMesh(core_axis_name="c", subcore_axis_name="s",
                                  num_cores=1)

    WS = 40

    @functools.partial(
        pl.kernel, mesh=mesh,
        out_type=jax.ShapeDtypeStruct((npad, A), F32),
        scratch_types=(
            [pltpu.VMEM_SHARED((npad, A), F32)]
            + [pltpu.VMEM((WS, EW), jnp.int32)]
            + [pltpu.VMEM((WS, EW), F32)]
            + [pltpu.VMEM((EW, A), F32)] * NB
            + [pltpu.VMEM((EW // 32, A), F32)] * NB
            + [pltpu.SemaphoreType.DMA] * (2 * NB)
        ),
    )
    def sc1(table_h, idx_h, w_h, out_h, shared, idx_v, w_v, *bufs):
        sid = lax.axis_index("s")
        wid = sid
        gbs = bufs[:NB]
        obs = bufs[NB:2 * NB]
        gsems = bufs[2 * NB:3 * NB]
        osems = bufs[3 * NB:4 * NB]
        _stage_to_spmem(table_h, shared, gbs[0], sid, npad // (16 * EW), EW)

        def visit(gt, lt, j):
            @pl.when(gt >= NB)
            def _():
                pltpu.make_async_copy(
                    obs[j], out_h.at[pl.ds(0, EW // 32)], osems[j]).wait()

            pltpu.make_async_copy(
                table_h.at[pl.ds(0, EW)], gbs[j], gsems[j]).wait()
            gbuf, obuf = gbs[j], obs[j]
            for r4 in range(EW // 32):
                acc = [jnp.zeros((16,), F32) for _ in range(A // 16)]
                for g in range(2):
                    wv = w_v[lt, pl.ds((r4 * 2 + g) * 16, 16)]
                    for ln in range(16):
                        wb = _bcast16(wv, ln)
                        e = r4 * 32 + g * 16 + ln
                        for c in range(A // 16):
                            acc[c] = acc[c] + wb * gbuf[e, pl.ds(c * 16, 16)]
                for c in range(A // 16):
                    obuf[r4, pl.ds(c * 16, 16)] = acc[c]
            pltpu.async_copy(
                obuf, out_h.at[pl.ds(wid * rpw + gt * (EW // 32), EW // 32)],
                osems[j])

            @pl.when(lt + NB < WS)
            def _():
                pltpu.async_copy(shared.at[idx_v.at[lt + NB]], gbs[j], gsems[j])

        def window(win, carry):
            pltpu.sync_copy(idx_h.at[wid, pl.ds(win * WS, WS)], idx_v)
            pltpu.sync_copy(w_h.at[wid, pl.ds(win * WS, WS)], w_v)
            for j in range(NB):
                pltpu.async_copy(shared.at[idx_v.at[j]], gbs[j], gsems[j])

            def group(q, c2):
                for j in range(NB):
                    lt = q * NB + j
                    visit(win * WS + lt, lt, j)
                return c2

            lax.fori_loop(0, WS // NB, group, 0)
            return carry

        lax.fori_loop(0, steps // WS, window, 0)
        for j in range(NB):
            pltpu.make_async_copy(
                obs[j], out_h.at[pl.ds(0, EW // 32)], osems[j]).wait()

    return sc1(table, idx_r, w_r)


def _sc2_call(table, idx_r, nrows_pad_q, steps):
    npad, D = table.shape
    epw_q = steps * 32
    NB = 2
    WS = 40
    mesh = plsc.VectorSubcoreMesh(core_axis_name="c", subcore_axis_name="s",
                                  num_cores=1)

    @functools.partial(
        pl.kernel, mesh=mesh,
        out_type=jax.ShapeDtypeStruct((nrows_pad_q, 128), F32),
        scratch_types=(
            [pltpu.VMEM_SHARED((npad, D), F32)]
            + [pltpu.VMEM((WS, 128), jnp.int32)]
            + [pltpu.VMEM((128, D), F32)] * NB
            + [pltpu.VMEM((32, 128), F32)] * NB
            + [pltpu.SemaphoreType.DMA] * (2 * NB)
        ),
    )
    def sc2(table_h, idx_h, out_h, shared, idx_v, *bufs):
        sid = lax.axis_index("s")
        wid = sid
        gbs = bufs[:NB]
        obs = bufs[NB:2 * NB]
        gsems = bufs[2 * NB:3 * NB]
        osems = bufs[3 * NB:4 * NB]
        _stage_to_spmem(table_h, shared, gbs[0], sid, npad // (16 * 128), 128)

        def visit(gt, lt, j):
            @pl.when(gt >= NB)
            def _():
                pltpu.make_async_copy(
                    obs[j], out_h.at[pl.ds(0, 32)], osems[j]).wait()

            pltpu.make_async_copy(
                table_h.at[pl.ds(0, 128)], gbs[j], gsems[j]).wait()
            gbuf, obuf = gbs[j], obs[j]
            for r in range(32):
                for k in range(4):
                    e = r * 4 + k
                    obuf[r, pl.ds(k * 32, 16)] = gbuf[e, pl.ds(0, 16)]
                    obuf[r, pl.ds(k * 32 + 16, 16)] = gbuf[e, pl.ds(16, 16)]
            pltpu.async_copy(
                obuf, out_h.at[pl.ds(wid * epw_q + gt * 32, 32)], osems[j])

            @pl.when(lt + NB < WS)
            def _():
                pltpu.async_copy(shared.at[idx_v.at[lt + NB]], gbs[j], gsems[j])

        def window(win, carry):
            pltpu.sync_copy(idx_h.at[wid, pl.ds(win * WS, WS)], idx_v)
            for j in range(NB):
                pltpu.async_copy(shared.at[idx_v.at[j]], gbs[j], gsems[j])

            def group(q, c2):
                for j in range(NB):
                    lt = q * NB + j
                    visit(win * WS + lt, lt, j)
                return c2

            lax.fori_loop(0, WS // NB, group, 0)
            return carry

        lax.fori_loop(0, steps // WS, window, 0)
        for j in range(NB):
            pltpu.make_async_copy(
                obs[j], out_h.at[pl.ds(0, 32)], osems[j]).wait()

    return sc2(table, idx_r)


def _k1_body(n_real, agi, agj, nbrf, af, expand, shrink,
             wag_o, sumw_o, p3_o, af_o):
    af_o[...] = af[...]
    w = agi[...] * agj[...]
    wag_o[...] = w
    sumw_o[...] = jnp.sum(w, axis=1, keepdims=True)
    we = jnp.dot(w, expand[...], preferred_element_type=F32)
    p3_o[...] = jnp.dot(nbrf[...] * we, shrink[...],
                        preferred_element_type=F32)


def _k234_body(af, g1, p3, sw, wct, wnt, wbt, bias, b1g, b1b, b2g, b2b,
               lng, lnb, bct, bnt, b0,
               ao_o, pc_o, pn_o,
               lin_s, y_s, s1, s2, a1, c1, t1, t2, a2, c2v):
    p = pl.program_id(0)
    i = pl.program_id(1)
    TN, A = af.shape
    N = lin_s.shape[0]
    row = pl.ds(i * TN, TN)

    @pl.when(p == 0)
    def _():
        x = af[...] * sw[...]
        lin = (jnp.dot(x, wct[...], preferred_element_type=F32)
               + jnp.dot(g1[...], wnt[...], preferred_element_type=F32)
               + jnp.dot(p3[...], wbt[...], preferred_element_type=F32)
               + bias[...])
        lin_s[row, :] = lin
        ls1 = jnp.sum(lin, axis=0, keepdims=True)
        ls2 = jnp.sum(lin * lin, axis=0, keepdims=True)

        @pl.when(i == 0)
        def _():
            s1[...] = ls1
            s2[...] = ls2

        @pl.when(i != 0)
        def _():
            s1[...] += ls1
            s2[...] += ls2

    @pl.when(p == 1)
    def _():
        @pl.when(i == 0)
        def _():
            mu = s1[...] / N
            var = s2[...] / N - mu * mu
            aa = b1g[...] * lax.rsqrt(var + _EPS)
            a1[...] = aa
            c1[...] = b1b[...] - mu * aa

        ln = lin_s[row, :] * a1[...] + c1[...]
        y = jax.nn.sigmoid(ln[:, :A]) * _softplus(ln[:, A:])
        y_s[row, :] = y
        ys1 = jnp.sum(y, axis=0, keepdims=True)
        ys2 = jnp.sum(y * y, axis=0, keepdims=True)

        @pl.when(i == 0)
        def _():
            t1[...] = ys1
            t2[...] = ys2

        @pl.when(i != 0)
        def _():
            t1[...] += ys1
            t2[...] += ys2

    @pl.when(p == 2)
    def _():
        @pl.when(i == 0)
        def _():
            mu = t1[...] / N
            var = t2[...] / N - mu * mu
            aa = b2g[...] * lax.rsqrt(var + _EPS)
            a2[...] = aa
            c2v[...] = b2b[...] - mu * aa

        z = _softplus(y_s[row, :] * a2[...] + c2v[...])
        t = z + af[...]
        mu = jnp.mean(t, axis=1, keepdims=True)
        d = t - mu
        var = jnp.mean(d * d, axis=1, keepdims=True)
        ao = d * lax.rsqrt(var + _EPS) * lng[...] + lnb[...]
        ao_o[...] = ao
        pc_o[...] = jnp.dot(ao, bct[...], preferred_element_type=F32) + b0[...]
        pn = jnp.dot(ao, bnt[...], preferred_element_type=F32)
        pn_o[...] = jnp.concatenate([pn, pn, pn, pn], axis=1)

def _k2_body(af, g1, p3, sw, wct, wnt, wbt, bias, lin_o, s1_o, s2_o):
    x = af[...] * sw[...]
    lin = (jnp.dot(x, wct[...], preferred_element_type=F32)
           + jnp.dot(g1[...], wnt[...], preferred_element_type=F32)
           + jnp.dot(p3[...], wbt[...], preferred_element_type=F32)
           + bias[...])
    lin_o[...] = lin
    s1 = jnp.sum(lin, axis=0, keepdims=True)
    s2 = jnp.sum(lin * lin, axis=0, keepdims=True)

    @pl.when(pl.program_id(0) == 0)
    def _():
        s1_o[...] = s1
        s2_o[...] = s2

    @pl.when(pl.program_id(0) != 0)
    def _():
        s1_o[...] += s1
        s2_o[...] += s2


def _k3_body(lin, a1, c1, y_o, s1_o, s2_o):
    A = y_o.shape[1]
    ln = lin[...] * a1[...] + c1[...]
    y = jax.nn.sigmoid(ln[:, :A]) * _softplus(ln[:, A:])
    y_o[...] = y
    s1 = jnp.sum(y, axis=0, keepdims=True)
    s2 = jnp.sum(y * y, axis=0, keepdims=True)

    @pl.when(pl.program_id(0) == 0)
    def _():
        s1_o[...] = s1
        s2_o[...] = s2

    @pl.when(pl.program_id(0) != 0)
    def _():
        s1_o[...] += s1
        s2_o[...] += s2


def _k4_body(y, af, a2, c2, lng, lnb, bct, bnt, b0, ao_o, pc_o, pn_o):
    z = _softplus(y[...] * a2[...] + c2[...])
    t = z + af[...]
    mu = jnp.mean(t, axis=1, keepdims=True)
    d = t - mu
    var = jnp.mean(d * d, axis=1, keepdims=True)
    ao = d * lax.rsqrt(var + _EPS) * lng[...] + lnb[...]
    ao_o[...] = ao
    pc_o[...] = jnp.dot(ao, bct[...], preferred_element_type=F32) + b0[...]
    pn = jnp.dot(ao, bnt[...], preferred_element_type=F32)
    pn_o[...] = jnp.concatenate([pn, pn, pn, pn], axis=1)


def _k6_body(pc, g2p, nbrp, bg4, bbt4, bu1t4, bu2t4, b14, b24, onesb, mavg,
             lng4, lnb4, out_o):
    TN = pc.shape[0]
    R = g2p.shape[0]
    pc4 = jnp.concatenate([pc[...]] * 4, axis=1)
    pcb = jnp.broadcast_to(pc4[:, None, :], (TN, 8, 128)).reshape(R, 128)
    bp = jnp.dot(nbrp[...], bbt4[...], preferred_element_type=F32)
    h0 = _silu(pcb + g2p[...].astype(F32) + bp)
    h1 = _silu(jnp.dot(h0, bu1t4[...], preferred_element_type=F32) + b14[...])
    h2 = jnp.dot(h1, bu2t4[...], preferred_element_type=F32) + b24[...]
    bgb = jnp.dot(bg4[...], onesb[...], preferred_element_type=F32)
    v = h2 * bgb + nbrp[...]
    mu = jnp.dot(v, mavg[...], preferred_element_type=F32)
    d = v - mu
    var = jnp.dot(d * d, mavg[...], preferred_element_type=F32)
    out_o[...] = d * lax.rsqrt(var + _EPS) * lng4[...] + lnb4[...]


def kernel(atom_fea, nbr_fea, nbr_fea_idx, bond_weights_ag_i,
           bond_weights_ag_j, bond_weights_bg_i, bond_weights_bg_j,
           fc_full_W, fc_full_b, bn1_g, bn1_b, bn2_g, bn2_b,
           ln_atom_g, ln_atom_b, bu0_W, bu0_b, bu1_W, bu1_b, bu2_W, bu2_b,
           ln_bond_g, ln_bond_b):
    Nn, Mm = nbr_fea_idx.shape
    A = atom_fea.shape[-1]
    B = nbr_fea.shape[-1]
    NW = 16
    NPAD = -(-Nn // 2048) * 2048
    RPW = NPAD // NW
    ST1 = RPW * Mm // 128
    ST2 = RPW * Mm // 128

    def to_worker(x, dtype, st, ew):
        xp = jnp.zeros((NPAD, Mm), dtype).at[:Nn].set(x)
        return xp.reshape(NW, st, ew)


    TN1 = RPW
    EWK = TN1 * Mm // 128
    nbrf2 = nbr_fea.reshape(Nn, Mm * B)
    import functools as _ft
    wag, sumw, p3, af_pad = pl.pallas_call(
        _ft.partial(_k1_body, Nn),
        grid=(NW,),
        in_specs=[
            pl.BlockSpec((TN1, Mm), lambda i: (i, 0)),
            pl.BlockSpec((TN1, Mm), lambda i: (i, 0)),
            pl.BlockSpec((TN1, Mm * B), lambda i: (i, 0)),
            pl.BlockSpec((TN1, A), lambda i: (i, 0)),
            pl.BlockSpec((Mm, Mm * B), lambda i: (0, 0)),
            pl.BlockSpec((Mm * B, B), lambda i: (0, 0)),
        ],
        out_specs=[
            pl.BlockSpec((TN1, Mm), lambda i: (i, 0)),
            pl.BlockSpec((TN1, 1), lambda i: (i, 0)),
            pl.BlockSpec((TN1, B), lambda i: (i, 0)),
            pl.BlockSpec((TN1, A), lambda i: (i, 0)),
        ],
        out_shape=[
            jax.ShapeDtypeStruct((NPAD, Mm), F32),
            jax.ShapeDtypeStruct((NPAD, 1), F32),
            jax.ShapeDtypeStruct((NPAD, B), F32),
            jax.ShapeDtypeStruct((NPAD, A), F32),
        ],
    )(bond_weights_ag_i, bond_weights_ag_j, nbrf2, atom_fea,
      jnp.kron(jnp.eye(Mm, dtype=F32), jnp.ones((1, B), F32)),
      jnp.kron(jnp.ones((Mm, 1), F32), jnp.eye(B, dtype=F32)))

    idx_rk = to_worker(nbr_fea_idx, jnp.int32, ST2, 128)
    w_r = wag.reshape(NW, ST1, 128)

    g1 = _sc1_call(af_pad, idx_rk, w_r, NPAD, RPW, ST1)

    TN = 400
    C2 = 2 * A
    PD = bu0_W.shape[0]
    wct = fc_full_W[:, :A].T
    wnt = fc_full_W[:, A:2 * A].T
    wbt = fc_full_W[:, 2 * A:].T
    bct = bu0_W[:, :A].T
    bnt = bu0_W[:, A:2 * A].T
    _b = lambda r, c: pl.BlockSpec((r, c), lambda p, i: (0, 0))
    atom_out, pc, pn = pl.pallas_call(
        _k234_body,
        grid=(3, Nn // TN),
        in_specs=[
            pl.BlockSpec((TN, A), lambda p, i: (i, 0)),
            pl.BlockSpec((TN, A), lambda p, i: (i, 0)),
            pl.BlockSpec((TN, B), lambda p, i: (i, 0)),
            pl.BlockSpec((TN, 1), lambda p, i: (i, 0)),
            _b(A, C2), _b(A, C2), _b(B, C2), _b(1, C2),
            _b(1, C2), _b(1, C2), _b(1, A), _b(1, A),
            _b(1, A), _b(1, A), _b(A, PD), _b(A, PD), _b(1, PD),
        ],
        out_specs=[
            pl.BlockSpec((TN, A), lambda p, i: (i, 0)),
            pl.BlockSpec((TN, PD), lambda p, i: (i, 0)),
            pl.BlockSpec((TN, 4 * PD), lambda p, i: (i, 0)),
        ],
        out_shape=[
            jax.ShapeDtypeStruct((Nn, A), F32),
            jax.ShapeDtypeStruct((Nn, PD), F32),
            jax.ShapeDtypeStruct((NPAD, 4 * PD), F32),
        ],
        scratch_shapes=[
            pltpu.VMEM((Nn, C2), F32),
            pltpu.VMEM((Nn, A), F32),
            pltpu.VMEM((1, C2), F32), pltpu.VMEM((1, C2), F32),
            pltpu.VMEM((1, C2), F32), pltpu.VMEM((1, C2), F32),
            pltpu.VMEM((1, A), F32), pltpu.VMEM((1, A), F32),
            pltpu.VMEM((1, A), F32), pltpu.VMEM((1, A), F32),
        ],
    )(atom_fea, g1, p3, sumw, wct, wnt, wbt, fc_full_b.reshape(1, C2),
      bn1_g.reshape(1, C2), bn1_b.reshape(1, C2),
      bn2_g.reshape(1, A), bn2_b.reshape(1, A),
      ln_atom_g.reshape(1, A), ln_atom_b.reshape(1, A),
      bct, bnt, bu0_b.reshape(1, PD))

    QPAD = NPAD * Mm // 4
    g2p = _sc2_call(pn, idx_rk, QPAD, ST2)

    TN6 = 200
    R6 = TN6 * Mm // 4
    Q = Nn * Mm // 4
    I4 = jnp.eye(4, dtype=F32)
    bbt4 = jnp.kron(I4, bu0_W[:, 2 * A:].T)
    bu1t4 = jnp.kron(I4, bu1_W.T)
    bu2t4 = jnp.kron(I4, bu2_W.T)
    onesb = jnp.kron(I4, jnp.ones((1, B), F32))
    mavg = jnp.kron(I4, jnp.full((B, B), 1.0 / B, F32))
    nbrp = nbr_fea.reshape(Q, 4 * B)
    bg4 = bond_weights_bg_i.reshape(Q, 4)
    nbr_out = pl.pallas_call(
        _k6_body,
        grid=(Nn // TN6,),
        in_specs=[
            pl.BlockSpec((TN6, PD), lambda i: (i, 0)),
            pl.BlockSpec((R6, 128), lambda i: (i, 0)),
            pl.BlockSpec((R6, 4 * B), lambda i: (i, 0)),
            pl.BlockSpec((R6, 4), lambda i: (i, 0)),
            pl.BlockSpec((4 * B, 128), lambda i: (0, 0)),
            pl.BlockSpec((128, 4 * B), lambda i: (0, 0)),
            pl.BlockSpec((4 * B, 4 * B), lambda i: (0, 0)),
            pl.BlockSpec((1, 4 * B), lambda i: (0, 0)),
            pl.BlockSpec((1, 4 * B), lambda i: (0, 0)),
            pl.BlockSpec((4, 4 * B), lambda i: (0, 0)),
            pl.BlockSpec((4 * B, 4 * B), lambda i: (0, 0)),
            pl.BlockSpec((1, 4 * B), lambda i: (0, 0)),
            pl.BlockSpec((1, 4 * B), lambda i: (0, 0)),
        ],
        out_specs=pl.BlockSpec((R6, 4 * B), lambda i: (i, 0)),
        out_shape=jax.ShapeDtypeStruct((Q, 4 * B), F32),
    )(pc, g2p, nbrp, bg4, bbt4, bu1t4, bu2t4,
      jnp.tile(bu1_b, 4).reshape(1, 4 * B), jnp.tile(bu2_b, 4).reshape(1, 4 * B),
      onesb, mavg,
      jnp.tile(ln_bond_g, 4).reshape(1, 4 * B),
      jnp.tile(ln_bond_b, 4).reshape(1, 4 * B))

    return (atom_out, nbr_out.reshape(Nn, Mm, B))


# speedup vs baseline: 1.0229x; 1.0229x over previous
import functools

import jax
import jax.numpy as jnp
from jax import lax
from jax.experimental import pallas as pl
from jax.experimental.pallas import tpu as pltpu
from jax.experimental.pallas import tpu_sc as plsc

F32 = jnp.float32
_EPS = 1e-5

_BCAST_DNUMS = lax.GatherDimensionNumbers(
    offset_dims=(), collapsed_slice_dims=(0,), start_index_map=(0,))


def _bcast16(v, lane):
    idx = jnp.full((16, 1), lane, jnp.int32)
    return lax.gather(v, idx, _BCAST_DNUMS, (1,),
                      indices_are_sorted=True, unique_indices=False,
                      mode=lax.GatherScatterMode.PROMISE_IN_BOUNDS)


def _softplus(x):
    return jnp.maximum(x, 0.0) + jnp.log1p(jnp.exp(-jnp.abs(x)))


def _silu(x):
    return x * jax.nn.sigmoid(x)


def _stage_to_spmem(table_h, shared, buf, sid, chunks, rows):
    for c in range(chunks):
        base = sid * (chunks * rows) + c * rows
        pltpu.sync_copy(table_h.at[pl.ds(base, rows)], buf)
        pltpu.sync_copy(buf, shared.at[pl.ds(base, rows)])
    plsc.subcore_barrier()


def _sc1_call(table, idx_r, w_r, npad, rpw, steps):
    A = table.shape[1]
    NB = 2
    EW = 128
    mesh = plsc.VectorSubcoreMesh(core_axis_name="c", subcore_axis_name="s",
                                  num_cores=1)

    WS = 40

    @functools.partial(
        pl.kernel, mesh=mesh,
        out_type=jax.ShapeDtypeStruct((npad, A), F32),
        scratch_types=(
            [pltpu.VMEM_SHARED((npad, A), F32)]
            + [pltpu.VMEM((WS, EW), jnp.int32)]
            + [pltpu.VMEM((WS, EW), F32)]
            + [pltpu.VMEM((EW, A), F32)] * NB
            + [pltpu.VMEM((EW // 32, A), F32)] * NB
            + [pltpu.SemaphoreType.DMA] * (2 * NB)
        ),
    )
    def sc1(table_h, idx_h, w_h, out_h, shared, idx_v, w_v, *bufs):
        sid = lax.axis_index("s")
        wid = sid
        gbs = bufs[:NB]
        obs = bufs[NB:2 * NB]
        gsems = bufs[2 * NB:3 * NB]
        osems = bufs[3 * NB:4 * NB]
        _stage_to_spmem(table_h, shared, gbs[0], sid, npad // (16 * EW), EW)

        def visit(gt, lt, j):
            @pl.when(gt >= NB)
            def _():
                pltpu.make_async_copy(
                    obs[j], out_h.at[pl.ds(0, EW // 32)], osems[j]).wait()

            pltpu.make_async_copy(
                table_h.at[pl.ds(0, EW)], gbs[j], gsems[j]).wait()
            gbuf, obuf = gbs[j], obs[j]
            for r4 in range(EW // 32):
                acc = [jnp.zeros((16,), F32) for _ in range(A // 16)]
                for g in range(2):
                    wv = w_v[lt, pl.ds((r4 * 2 + g) * 16, 16)]
                    for ln in range(16):
                        wb = _bcast16(wv, ln)
                        e = r4 * 32 + g * 16 + ln
                        for c in range(A // 16):
                            acc[c] = acc[c] + wb * gbuf[e, pl.ds(c * 16, 16)]
                for c in range(A // 16):
                    obuf[r4, pl.ds(c * 16, 16)] = acc[c]
            pltpu.async_copy(
                obuf, out_h.at[pl.ds(wid * rpw + gt * (EW // 32), EW // 32)],
                osems[j])

            @pl.when(lt + NB < WS)
            def _():
                pltpu.async_copy(shared.at[idx_v.at[lt + NB]], gbs[j], gsems[j])

        def window(win, carry):
            pltpu.sync_copy(idx_h.at[wid, pl.ds(win * WS, WS)], idx_v)
            pltpu.sync_copy(w_h.at[wid, pl.ds(win * WS, WS)], w_v)
            for j in range(NB):
                pltpu.async_copy(shared.at[idx_v.at[j]], gbs[j], gsems[j])

            def group(q, c2):
                for j in range(NB):
                    lt = q * NB + j
                    visit(win * WS + lt, lt, j)
                return c2

            lax.fori_loop(0, WS // NB, group, 0)
            return carry

        lax.fori_loop(0, steps // WS, window, 0)
        for j in range(NB):
            pltpu.make_async_copy(
                obs[j], out_h.at[pl.ds(0, EW // 32)], osems[j]).wait()

    return sc1(table, idx_r, w_r)


def _sc2_call(table, idx_r, nrows_pad_q, steps):
    npad, D = table.shape
    epw_q = steps * 32
    NB = 2
    WS = 40
    mesh = plsc.VectorSubcoreMesh(core_axis_name="c", subcore_axis_name="s",
                                  num_cores=1)

    @functools.partial(
        pl.kernel, mesh=mesh,
        out_type=jax.ShapeDtypeStruct((nrows_pad_q, 128), F32),
        scratch_types=(
            [pltpu.VMEM_SHARED((npad, D), F32)]
            + [pltpu.VMEM((WS, 128), jnp.int32)]
            + [pltpu.VMEM((128, D), F32)] * NB
            + [pltpu.VMEM((32, 128), F32)] * NB
            + [pltpu.SemaphoreType.DMA] * (2 * NB)
        ),
    )
    def sc2(table_h, idx_h, out_h, shared, idx_v, *bufs):
        sid = lax.axis_index("s")
        wid = sid
        gbs = bufs[:NB]
        obs = bufs[NB:2 * NB]
        gsems = bufs[2 * NB:3 * NB]
        osems = bufs[3 * NB:4 * NB]
        _stage_to_spmem(table_h, shared, gbs[0], sid, npad // (16 * 128), 128)

        def visit(gt, lt, j):
            @pl.when(gt >= NB)
            def _():
                pltpu.make_async_copy(
                    obs[j], out_h.at[pl.ds(0, 32)], osems[j]).wait()

            pltpu.make_async_copy(
                table_h.at[pl.ds(0, 128)], gbs[j], gsems[j]).wait()
            gbuf, obuf = gbs[j], obs[j]
            for r in range(32):
                for k in range(4):
                    e = r * 4 + k
                    obuf[r, pl.ds(k * 32, 16)] = gbuf[e, pl.ds(0, 16)]
                    obuf[r, pl.ds(k * 32 + 16, 16)] = gbuf[e, pl.ds(16, 16)]
            pltpu.async_copy(
                obuf, out_h.at[pl.ds(wid * epw_q + gt * 32, 32)], osems[j])

            @pl.when(lt + NB < WS)
            def _():
                pltpu.async_copy(shared.at[idx_v.at[lt + NB]], gbs[j], gsems[j])

        def window(win, carry):
            pltpu.sync_copy(idx_h.at[wid, pl.ds(win * WS, WS)], idx_v)
            for j in range(NB):
                pltpu.async_copy(shared.at[idx_v.at[j]], gbs[j], gsems[j])

            def group(q, c2):
                for j in range(NB):
                    lt = q * NB + j
                    visit(win * WS + lt, lt, j)
                return c2

            lax.fori_loop(0, WS // NB, group, 0)
            return carry

        lax.fori_loop(0, steps // WS, window, 0)
        for j in range(NB):
            pltpu.make_async_copy(
                obs[j], out_h.at[pl.ds(0, 32)], osems[j]).wait()

    return sc2(table, idx_r)


def _k1_body(n_real, agi, agj, nbrf, af, expand, shrink,
             wag_o, sumw_o, p3_o, af_o):
    af_o[...] = af[...]
    w = agi[...] * agj[...]
    wag_o[...] = w
    sumw_o[...] = jnp.sum(w, axis=1, keepdims=True)
    we = jnp.dot(w, expand[...], preferred_element_type=F32)
    p3_o[...] = jnp.dot(nbrf[...] * we, shrink[...],
                        preferred_element_type=F32)


def _k234_body(af, g1, p3, sw, wct, wnt, wbt, bias, b1g, b1b, b2g, b2b,
               lng, lnb, bct, bnt, b0,
               ao_o, pc_o, pn_o,
               lin_s, y_s, s1, s2, a1, c1, t1, t2, a2, c2v):
    p = pl.program_id(0)
    i = pl.program_id(1)
    TN, A = af.shape
    N = lin_s.shape[0]
    row = pl.ds(i * TN, TN)

    @pl.when(p == 0)
    def _():
        x = af[...] * sw[...]
        lin = (jnp.dot(x, wct[...], preferred_element_type=F32)
               + jnp.dot(g1[...], wnt[...], preferred_element_type=F32)
               + jnp.dot(p3[...], wbt[...], preferred_element_type=F32)
               + bias[...])
        lin_s[row, :] = lin
        ls1 = jnp.sum(lin, axis=0, keepdims=True)
        ls2 = jnp.sum(lin * lin, axis=0, keepdims=True)

        @pl.when(i == 0)
        def _():
            s1[...] = ls1
            s2[...] = ls2

        @pl.when(i != 0)
        def _():
            s1[...] += ls1
            s2[...] += ls2

    @pl.when(p == 1)
    def _():
        @pl.when(i == 0)
        def _():
            mu = s1[...] / N
            var = s2[...] / N - mu * mu
            aa = b1g[...] * lax.rsqrt(var + _EPS)
            a1[...] = aa
            c1[...] = b1b[...] - mu * aa

        ln = lin_s[row, :] * a1[...] + c1[...]
        y = jax.nn.sigmoid(ln[:, :A]) * _softplus(ln[:, A:])
        y_s[row, :] = y
        ys1 = jnp.sum(y, axis=0, keepdims=True)
        ys2 = jnp.sum(y * y, axis=0, keepdims=True)

        @pl.when(i == 0)
        def _():
            t1[...] = ys1
            t2[...] = ys2

        @pl.when(i != 0)
        def _():
            t1[...] += ys1
            t2[...] += ys2

    @pl.when(p == 2)
    def _():
        @pl.when(i == 0)
        def _():
            mu = t1[...] / N
            var = t2[...] / N - mu * mu
            aa = b2g[...] * lax.rsqrt(var + _EPS)
            a2[...] = aa
            c2v[...] = b2b[...] - mu * aa

        z = _softplus(y_s[row, :] * a2[...] + c2v[...])
        t = z + af[...]
        mu = jnp.mean(t, axis=1, keepdims=True)
        d = t - mu
        var = jnp.mean(d * d, axis=1, keepdims=True)
        ao = d * lax.rsqrt(var + _EPS) * lng[...] + lnb[...]
        ao_o[...] = ao
        pc_o[...] = jnp.dot(ao, bct[...], preferred_element_type=F32) + b0[...]
        pn = jnp.dot(ao, bnt[...], preferred_element_type=F32)
        pn_o[...] = jnp.concatenate([pn, pn, pn, pn], axis=1)

def _k2_body(af, g1, p3, sw, wct, wnt, wbt, bias, lin_o, s1_o, s2_o):
    x = af[...] * sw[...]
    lin = (jnp.dot(x, wct[...], preferred_element_type=F32)
           + jnp.dot(g1[...], wnt[...], preferred_element_type=F32)
           + jnp.dot(p3[...], wbt[...], preferred_element_type=F32)
           + bias[...])
    lin_o[...] = lin
    s1 = jnp.sum(lin, axis=0, keepdims=True)
    s2 = jnp.sum(lin * lin, axis=0, keepdims=True)

    @pl.when(pl.program_id(0) == 0)
    def _():
        s1_o[...] = s1
        s2_o[...] = s2

    @pl.when(pl.program_id(0) != 0)
    def _():
        s1_o[...] += s1
        s2_o[...] += s2


def _k3_body(lin, a1, c1, y_o, s1_o, s2_o):
    A = y_o.shape[1]
    ln = lin[...] * a1[...] + c1[...]
    y = jax.nn.sigmoid(ln[:, :A]) * _softplus(ln[:, A:])
    y_o[...] = y
    s1 = jnp.sum(y, axis=0, keepdims=True)
    s2 = jnp.sum(y * y, axis=0, keepdims=True)

    @pl.when(pl.program_id(0) == 0)
    def _():
        s1_o[...] = s1
        s2_o[...] = s2

    @pl.when(pl.program_id(0) != 0)
    def _():
        s1_o[...] += s1
        s2_o[...] += s2


def _k4_body(y, af, a2, c2, lng, lnb, bct, bnt, b0, ao_o, pc_o, pn_o):
    z = _softplus(y[...] * a2[...] + c2[...])
    t = z + af[...]
    mu = jnp.mean(t, axis=1, keepdims=True)
    d = t - mu
    var = jnp.mean(d * d, axis=1, keepdims=True)
    ao = d * lax.rsqrt(var + _EPS) * lng[...] + lnb[...]
    ao_o[...] = ao
    pc_o[...] = jnp.dot(ao, bct[...], preferred_element_type=F32) + b0[...]
    pn = jnp.dot(ao, bnt[...], preferred_element_type=F32)
    pn_o[...] = jnp.concatenate([pn, pn, pn, pn], axis=1)


def _k6_body(pc, g2p, nbrp, bg4, bbt4, bu1t4, bu2t4, b14, b24, onesb, mavg,
             lng4, lnb4, out_o):
    TN = pc.shape[0]
    R = g2p.shape[0]
    pc4 = jnp.concatenate([pc[...]] * 4, axis=1)
    pcb = jnp.broadcast_to(pc4[:, None, :], (TN, 8, 128)).reshape(R, 128)
    bp = jnp.dot(nbrp[...], bbt4[...], preferred_element_type=F32)
    h0 = _silu(pcb + g2p[...].astype(F32) + bp)
    h1 = _silu(jnp.dot(h0, bu1t4[...], preferred_element_type=F32) + b14[...])
    h2 = jnp.dot(h1, bu2t4[...], preferred_element_type=F32) + b24[...]
    bgb = jnp.dot(bg4[...], onesb[...], preferred_element_type=F32)
    v = h2 * bgb + nbrp[...]
    mu = jnp.dot(v, mavg[...], preferred_element_type=F32)
    d = v - mu
    var = jnp.dot(d * d, mavg[...], preferred_element_type=F32)
    out_o[...] = d * lax.rsqrt(var + _EPS) * lng4[...] + lnb4[...]


def kernel(atom_fea, nbr_fea, nbr_fea_idx, bond_weights_ag_i,
           bond_weights_ag_j, bond_weights_bg_i, bond_weights_bg_j,
           fc_full_W, fc_full_b, bn1_g, bn1_b, bn2_g, bn2_b,
           ln_atom_g, ln_atom_b, bu0_W, bu0_b, bu1_W, bu1_b, bu2_W, bu2_b,
           ln_bond_g, ln_bond_b):
    Nn, Mm = nbr_fea_idx.shape
    A = atom_fea.shape[-1]
    B = nbr_fea.shape[-1]
    NW = 16
    NPAD = -(-Nn // 2048) * 2048
    RPW = NPAD // NW
    ST1 = RPW * Mm // 128
    ST2 = RPW * Mm // 128

    def to_worker(x, dtype, st, ew):
        xp = jnp.zeros((NPAD, Mm), dtype).at[:Nn].set(x)
        return xp.reshape(NW, st, ew)


    TN1 = RPW
    EWK = TN1 * Mm // 128
    nbrf2 = nbr_fea.reshape(Nn, Mm * B)
    import functools as _ft
    wag, sumw, p3, af_pad = pl.pallas_call(
        _ft.partial(_k1_body, Nn),
        grid=(NW,),
        in_specs=[
            pl.BlockSpec((TN1, Mm), lambda i: (i, 0)),
            pl.BlockSpec((TN1, Mm), lambda i: (i, 0)),
            pl.BlockSpec((TN1, Mm * B), lambda i: (i, 0)),
            pl.BlockSpec((TN1, A), lambda i: (i, 0)),
            pl.BlockSpec((Mm, Mm * B), lambda i: (0, 0)),
            pl.BlockSpec((Mm * B, B), lambda i: (0, 0)),
        ],
        out_specs=[
            pl.BlockSpec((TN1, Mm), lambda i: (i, 0)),
            pl.BlockSpec((TN1, 1), lambda i: (i, 0)),
            pl.BlockSpec((TN1, B), lambda i: (i, 0)),
            pl.BlockSpec((TN1, A), lambda i: (i, 0)),
        ],
        out_shape=[
            jax.ShapeDtypeStruct((NPAD, Mm), F32),
            jax.ShapeDtypeStruct((NPAD, 1), F32),
            jax.ShapeDtypeStruct((NPAD, B), F32),
            jax.ShapeDtypeStruct((NPAD, A), F32),
        ],
    )(bond_weights_ag_i, bond_weights_ag_j, nbrf2, atom_fea,
      jnp.kron(jnp.eye(Mm, dtype=F32), jnp.ones((1, B), F32)),
      jnp.kron(jnp.ones((Mm, 1), F32), jnp.eye(B, dtype=F32)))

    idx_rk = to_worker(nbr_fea_idx, jnp.int32, ST2, 128)
    w_r = wag.reshape(NW, ST1, 128)

    g1 = _sc1_call(af_pad, idx_rk, w_r, NPAD, RPW, ST1)

    TN = 400
    C2 = 2 * A
    PD = bu0_W.shape[0]
    wct = fc_full_W[:, :A].T
    wnt = fc_full_W[:, A:2 * A].T
    wbt = fc_full_W[:, 2 * A:].T
    bct = bu0_W[:, :A].T
    bnt = bu0_W[:, A:2 * A].T
    _b = lambda r, c: pl.BlockSpec((r, c), lambda p, i: (0, 0))
    atom_out, pc, pn = pl.pallas_call(
        _k234_body,
        grid=(3, Nn // TN),
        in_specs=[
            pl.BlockSpec((TN, A), lambda p, i: (i, 0)),
            pl.BlockSpec((TN, A), lambda p, i: (i, 0)),
            pl.BlockSpec((TN, B), lambda p, i: (i, 0)),
            pl.BlockSpec((TN, 1), lambda p, i: (i, 0)),
            _b(A, C2), _b(A, C2), _b(B, C2), _b(1, C2),
            _b(1, C2), _b(1, C2), _b(1, A), _b(1, A),
            _b(1, A), _b(1, A), _b(A, PD), _b(A, PD), _b(1, PD),
        ],
        out_specs=[
            pl.BlockSpec((TN, A), lambda p, i: (i, 0)),
            pl.BlockSpec((TN, PD), lambda p, i: (i, 0)),
            pl.BlockSpec((TN, 4 * PD), lambda p, i: (i, 0)),
        ],
        out_shape=[
            jax.ShapeDtypeStruct((Nn, A), F32),
            jax.ShapeDtypeStruct((Nn, PD), F32),
            jax.ShapeDtypeStruct((NPAD, 4 * PD), F32),
        ],
        scratch_shapes=[
            pltpu.VMEM((Nn, C2), F32),
            pltpu.VMEM((Nn, A), F32),
            pltpu.VMEM((1, C2), F32), pltpu.VMEM((1, C2), F32),
            pltpu.VMEM((1, C2), F32), pltpu.VMEM((1, C2), F32),
            pltpu.VMEM((1, A), F32), pltpu.VMEM((1, A), F32),
            pltpu.VMEM((1, A), F32), pltpu.VMEM((1, A), F32),
        ],
    )(atom_fea, g1, p3, sumw, wct, wnt, wbt, fc_full_b.reshape(1, C2),
      bn1_g.reshape(1, C2), bn1_b.reshape(1, C2),
      bn2_g.reshape(1, A), bn2_b.reshape(1, A),
      ln_atom_g.reshape(1, A), ln_atom_b.reshape(1, A),
      bct, bnt, bu0_b.reshape(1, PD))

    QPAD = NPAD * Mm // 4
    g2p = _sc2_call(pn, idx_rk, QPAD, ST2)

    TN6 = 400
    R6 = TN6 * Mm // 4
    Q = Nn * Mm // 4
    I4 = jnp.eye(4, dtype=F32)
    bbt4 = jnp.kron(I4, bu0_W[:, 2 * A:].T)
    bu1t4 = jnp.kron(I4, bu1_W.T)
    bu2t4 = jnp.kron(I4, bu2_W.T)
    onesb = jnp.kron(I4, jnp.ones((1, B), F32))
    mavg = jnp.kron(I4, jnp.full((B, B), 1.0 / B, F32))
    nbrp = nbr_fea.reshape(Q, 4 * B)
    bg4 = bond_weights_bg_i.reshape(Q, 4)
    nbr_out = pl.pallas_call(
        _k6_body,
        grid=(Nn // TN6,),
        in_specs=[
            pl.BlockSpec((TN6, PD), lambda i: (i, 0)),
            pl.BlockSpec((R6, 128), lambda i: (i, 0)),
            pl.BlockSpec((R6, 4 * B), lambda i: (i, 0)),
            pl.BlockSpec((R6, 4), lambda i: (i, 0)),
            pl.BlockSpec((4 * B, 128), lambda i: (0, 0)),
            pl.BlockSpec((128, 4 * B), lambda i: (0, 0)),
            pl.BlockSpec((4 * B, 4 * B), lambda i: (0, 0)),
            pl.BlockSpec((1, 4 * B), lambda i: (0, 0)),
            pl.BlockSpec((1, 4 * B), lambda i: (0, 0)),
            pl.BlockSpec((4, 4 * B), lambda i: (0, 0)),
            pl.BlockSpec((4 * B, 4 * B), lambda i: (0, 0)),
            pl.BlockSpec((1, 4 * B), lambda i: (0, 0)),
            pl.BlockSpec((1, 4 * B), lambda i: (0, 0)),
        ],
        out_specs=pl.BlockSpec((R6, 4 * B), lambda i: (i, 0)),
        out_shape=jax.ShapeDtypeStruct((Q, 4 * B), F32),
    )(pc, g2p, nbrp, bg4, bbt4, bu1t4, bu2t4,
      jnp.tile(bu1_b, 4).reshape(1, 4 * B), jnp.tile(bu2_b, 4).reshape(1, 4 * B),
      onesb, mavg,
      jnp.tile(ln_bond_g, 4).reshape(1, 4 * B),
      jnp.tile(ln_bond_b, 4).reshape(1, 4 * B))

    return (atom_out, nbr_out.reshape(Nn, Mm, B))
